# Initial kernel scaffold; baseline (speedup 1.0000x reference)
import jax
import jax.numpy as jnp
from jax.experimental import pallas as pl


def kernel(x, W_cheb, b_cheb, W_ih, W_hh, b_ih, b_hh, W_fc, b_fc):
    raise NotImplementedError("write your pallas kernel here")


# speedup vs baseline: 1.6376x; 1.6376x over previous
import jax
import jax.numpy as jnp
from jax.experimental import pallas as pl
from jax.experimental.pallas import tpu as pltpu


def _cheb_body(mid_lo, mid_hi):
    def body(xr_ref, w0_ref, w1_ref, bb_ref, out_ref, a_ref):
        xb = xr_ref[0]
        g = xb[:, mid_lo:mid_hi]
        sq = jnp.sum(g * g, axis=1, keepdims=True)
        gg = jax.lax.dot_general(
            g, g, (((1,), (1,)), ((), ())),
            preferred_element_type=jnp.float32)
        d2 = sq + jnp.transpose(sq) - 2.0 * gg
        a_ref[...] = jnp.exp(-jnp.maximum(d2, 0.0))
        deg = jnp.sum(a_ref[...], axis=1, keepdims=True)
        dinv = jax.lax.rsqrt(deg + 1e-6)
        y = dinv * xb
        tx1 = dinv * jnp.dot(a_ref[...], y,
                             preferred_element_type=jnp.float32)
        out = (jnp.dot(xb, w0_ref[...], preferred_element_type=jnp.float32)
               + jnp.dot(tx1, w1_ref[...], preferred_element_type=jnp.float32)
               + bb_ref[...])
        out_ref[0] = jnp.maximum(out, 0.0)
    return body


def _gru_body(n_k, n_t, n_b, hid):
    def body(gf_ref, wih_ref, bih_ref, whh_ref, bhh_ref, wfc_ref, bfc_ref,
             out_ref, acc_ref):
        k = pl.program_id(0)

        @pl.when(k == 0)
        def _init():
            acc_ref[...] = jnp.zeros_like(acc_ref)

        acc_ref[...] += jnp.dot(gf_ref[...], wih_ref[...],
                                preferred_element_type=jnp.float32)

        @pl.when(k == n_k - 1)
        def _finish():
            gx = acc_ref[...] + bih_ref[...]
            whh = whh_ref[...]
            bhh = bhh_ref[...]
            h = jnp.zeros((n_b, hid), dtype=jnp.float32)
            for t in range(n_t):
                gxt = gx[t * n_b:(t + 1) * n_b, :]
                gh = jnp.dot(h, whh,
                             preferred_element_type=jnp.float32) + bhh
                r = jax.nn.sigmoid(gxt[:, :hid] + gh[:, :hid])
                z = jax.nn.sigmoid(gxt[:, hid:2 * hid] + gh[:, hid:2 * hid])
                n = jnp.tanh(gxt[:, 2 * hid:] + r * gh[:, 2 * hid:])
                h = (1.0 - z) * n + z * h
            out_ref[...] = jnp.dot(h, wfc_ref[...],
                                   preferred_element_type=jnp.float32) \
                + bfc_ref[...]
    return body


def kernel(x, W_cheb, b_cheb, W_ih, W_hh, b_ih, b_hh, W_fc, b_fc):
    B, T, N, F = x.shape
    O = W_cheb.shape[-1]
    TF = T * F
    TO = T * O
    HID = W_hh.shape[0]
    HOUT = W_fc.shape[-1]
    D = N * O
    mid = T // 2

    xr = x.transpose(0, 2, 1, 3).reshape(B, N, TF)
    eyeT = jnp.eye(T, dtype=x.dtype)
    w0b = jnp.kron(eyeT, W_cheb[0])
    w1b = jnp.kron(eyeT, W_cheb[1])
    bb = jnp.tile(b_cheb, T)[None, :]

    cheb = pl.pallas_call(
        _cheb_body(mid * F, (mid + 1) * F),
        grid=(B,),
        in_specs=[
            pl.BlockSpec((1, N, TF), lambda b: (b, 0, 0)),
            pl.BlockSpec((TF, TO), lambda b: (0, 0)),
            pl.BlockSpec((TF, TO), lambda b: (0, 0)),
            pl.BlockSpec((1, TO), lambda b: (0, 0)),
        ],
        out_specs=pl.BlockSpec((1, N, TO), lambda b: (b, 0, 0)),
        out_shape=jax.ShapeDtypeStruct((B, N, TO), jnp.float32),
        scratch_shapes=[pltpu.VMEM((N, N), jnp.float32)],
    )
    G = cheb(xr, w0b, w1b, bb)

    Gf = G.reshape(B, N, T, O).transpose(2, 0, 1, 3).reshape(T * B, D)

    KB = 2048
    n_k = D // KB
    gru = pl.pallas_call(
        _gru_body(n_k, T, B, HID),
        grid=(n_k,),
        in_specs=[
            pl.BlockSpec((T * B, KB), lambda k: (0, k)),
            pl.BlockSpec((KB, 3 * HID), lambda k: (k, 0)),
            pl.BlockSpec((1, 3 * HID), lambda k: (0, 0)),
            pl.BlockSpec((HID, 3 * HID), lambda k: (0, 0)),
            pl.BlockSpec((1, 3 * HID), lambda k: (0, 0)),
            pl.BlockSpec((HID, HOUT), lambda k: (0, 0)),
            pl.BlockSpec((1, HOUT), lambda k: (0, 0)),
        ],
        out_specs=pl.BlockSpec((B, HOUT), lambda k: (0, 0)),
        out_shape=jax.ShapeDtypeStruct((B, HOUT), jnp.float32),
        scratch_shapes=[pltpu.VMEM((T * B, 3 * HID), jnp.float32)],
    )
    return gru(Gf, W_ih, b_ih[None, :], W_hh, b_hh[None, :],
               W_fc, b_fc[None, :])
